# baseline (speedup 1.0000x reference)
import functools

import jax
import jax.numpy as jnp
from jax import lax
from jax.experimental import pallas as pl
from jax.experimental.pallas import tpu as pltpu
from jax.experimental.pallas import tpu_sc as plsc

N = 100000
E = 6400000
LN = 128
R = E // LN
NC, NS = 2, 16
NW = NC * NS
K = 16
C = 98
RPW = K * C
R_PAD = NW * RPW
H = 102400
STRIPE = H // NS
BR = 2048
GRID = H // BR

_mesh = plsc.VectorSubcoreMesh(
    core_axis_name="c", subcore_axis_name="s", num_cores=NC, num_subcores=NS
)
_sc_params = pltpu.CompilerParams(use_tc_tiling_on_sc=False)


@functools.partial(
    pl.kernel,
    out_type=jax.ShapeDtypeStruct((NC, H, 1), jnp.float32),
    mesh=_mesh,
    scratch_types=[
        pltpu.VMEM((K, LN), jnp.int32),
        pltpu.VMEM((LN, 1), jnp.float32),
        pltpu.VMEM_SHARED((H, 1), jnp.float32),
        pltpu.SemaphoreType.DMA,
    ],
    compiler_params=_sc_params,
)
def _deg_kernel(dst_hbm, ones_hbm, zeros_hbm, out_hbm, didx, ones_v, hist_sh, sem):
    c = lax.axis_index("c")
    s = lax.axis_index("s")
    wid = c * NS + s
    sl = pl.ds(s * STRIPE, STRIPE)
    pltpu.sync_copy(ones_hbm, ones_v)
    pltpu.sync_copy(zeros_hbm.at[sl], hist_sh.at[sl])
    plsc.subcore_barrier()
    base = wid * RPW

    @pl.loop(0, C)
    def _chunk(ci):
        row0 = base + ci * K
        pltpu.sync_copy(dst_hbm.at[pl.ds(row0, K)], didx)
        descs = [
            pltpu.async_copy(ones_v, hist_sh.at[didx.at[j]], sem, add=True)
            for j in range(K)
        ]
        for d in descs:
            d.wait()

    plsc.subcore_barrier()
    pltpu.sync_copy(hist_sh.at[sl], out_hbm.at[c, sl])


def _make_agg(F):
    @functools.partial(
        pl.kernel,
        out_type=jax.ShapeDtypeStruct((NC, H, F), jnp.float32),
        mesh=_mesh,
        scratch_types=[
            pltpu.VMEM((K, LN), jnp.int32),
            pltpu.VMEM((K, LN), jnp.int32),
            pltpu.VMEM((K, LN, F), jnp.float32),
            pltpu.VMEM_SHARED((H, F), jnp.float32),
            pltpu.VMEM_SHARED((H, F), jnp.float32),
            pltpu.SemaphoreType.DMA,
            pltpu.SemaphoreType.DMA,
        ],
        compiler_params=_sc_params,
    )
    def _agg(src_hbm, dst_hbm, y_hbm, zeros_hbm, out_hbm,
             sidx, didx, rows_v, y_sh, agg_sh, sem_g, sem_s):
        c = lax.axis_index("c")
        s = lax.axis_index("s")
        wid = c * NS + s
        sl = pl.ds(s * STRIPE, STRIPE)
        pltpu.sync_copy(y_hbm.at[sl], y_sh.at[sl])
        pltpu.sync_copy(zeros_hbm.at[sl], agg_sh.at[sl])
        plsc.subcore_barrier()
        base = wid * RPW

        @pl.loop(0, C)
        def _chunk(ci):
            row0 = base + ci * K
            pltpu.sync_copy(src_hbm.at[pl.ds(row0, K)], sidx)
            pltpu.sync_copy(dst_hbm.at[pl.ds(row0, K)], didx)
            gd = [
                pltpu.async_copy(y_sh.at[sidx.at[j]], rows_v.at[j], sem_g)
                for j in range(K)
            ]
            for d in gd:
                d.wait()
            sd = [
                pltpu.async_copy(rows_v.at[j], agg_sh.at[didx.at[j]], sem_s, add=True)
                for j in range(K)
            ]
            for d in sd:
                d.wait()

        plsc.subcore_barrier()
        pltpu.sync_copy(agg_sh.at[sl], out_hbm.at[c, sl])

    return _agg


_agg2 = _make_agg(2)
_agg4 = _make_agg(4)


def _tc1_body(degp_ref, x_ref, dinv_ref, y1_ref):
    d = degp_ref[...]
    deg = d[0] + d[1] + 1.0
    dinv = lax.rsqrt(deg)
    dinv_ref[...] = dinv
    y1_ref[...] = x_ref[...] * dinv


_tc1 = pl.pallas_call(
    _tc1_body,
    grid=(GRID,),
    in_specs=[
        pl.BlockSpec((NC, BR, 1), lambda i: (0, i, 0)),
        pl.BlockSpec((BR, 2), lambda i: (i, 0)),
    ],
    out_specs=[
        pl.BlockSpec((BR, 1), lambda i: (i, 0)),
        pl.BlockSpec((BR, 2), lambda i: (i, 0)),
    ],
    out_shape=[
        jax.ShapeDtypeStruct((H, 1), jnp.float32),
        jax.ShapeDtypeStruct((H, 2), jnp.float32),
    ],
)


def _tc2_body(aggp_ref, y1_ref, dinv_ref, w1_ref, b1_ref, w2_ref, u_ref):
    a = aggp_ref[...]
    a1 = a[0] + a[1] + y1_ref[...]
    dinv = dinv_ref[...]
    t1 = a1 * dinv
    w1 = w1_ref[...]
    h = t1[:, 0:1] * w1[0:1, :] + t1[:, 1:2] * w1[1:2, :] + b1_ref[...]
    h = jnp.maximum(h, 0.0)
    w2 = w2_ref[...]
    z = (h[:, 0:1] * w2[0:1, :] + h[:, 1:2] * w2[1:2, :]
         + h[:, 2:3] * w2[2:3, :] + h[:, 3:4] * w2[3:4, :])
    u = z * dinv
    i = pl.program_id(0)
    rows = i * BR + lax.broadcasted_iota(jnp.int32, (BR, 1), 0)
    u_ref[...] = jnp.where(rows < N, u, 0.0)


_tc2 = pl.pallas_call(
    _tc2_body,
    grid=(GRID,),
    in_specs=[
        pl.BlockSpec((NC, BR, 2), lambda i: (0, i, 0)),
        pl.BlockSpec((BR, 2), lambda i: (i, 0)),
        pl.BlockSpec((BR, 1), lambda i: (i, 0)),
        pl.BlockSpec((2, 4), lambda i: (0, 0)),
        pl.BlockSpec((1, 4), lambda i: (0, 0)),
        pl.BlockSpec((4, 4), lambda i: (0, 0)),
    ],
    out_specs=pl.BlockSpec((BR, 4), lambda i: (i, 0)),
    out_shape=jax.ShapeDtypeStruct((H, 4), jnp.float32),
)


def _tc3_body(aggp_ref, u_ref, dinv_ref, b2_ref, w3_ref, b3_ref, w4_ref,
              b4_ref, w5_ref, b5_ref, o_ref):
    a = aggp_ref[...]
    a2 = a[0] + a[1] + u_ref[...]
    t2 = a2[:, 0:3] * dinv_ref[...] + b2_ref[...]
    g = 1.0 / (1.0 + jnp.exp(-t2))
    w3 = w3_ref[...]
    h3 = (g[:, 0:1] * w3[0:1, :] + g[:, 1:2] * w3[1:2, :]
          + g[:, 2:3] * w3[2:3, :] + b3_ref[...])
    h3 = jnp.maximum(h3, 0.0)
    w4 = w4_ref[...]
    h4 = (h3[:, 0:1] * w4[0:1, :] + h3[:, 1:2] * w4[1:2, :]
          + h3[:, 2:3] * w4[2:3, :] + h3[:, 3:4] * w4[3:4, :] + b4_ref[...])
    h4 = jnp.maximum(h4, 0.0)
    w5 = w5_ref[...]
    o_ref[...] = (h4[:, 0:1] * w5[0:1, :] + h4[:, 1:2] * w5[1:2, :]
                  + h4[:, 2:3] * w5[2:3, :] + b5_ref[...])


_tc3 = pl.pallas_call(
    _tc3_body,
    grid=(GRID,),
    in_specs=[
        pl.BlockSpec((NC, BR, 4), lambda i: (0, i, 0)),
        pl.BlockSpec((BR, 4), lambda i: (i, 0)),
        pl.BlockSpec((BR, 1), lambda i: (i, 0)),
        pl.BlockSpec((1, 3), lambda i: (0, 0)),
        pl.BlockSpec((3, 4), lambda i: (0, 0)),
        pl.BlockSpec((1, 4), lambda i: (0, 0)),
        pl.BlockSpec((4, 3), lambda i: (0, 0)),
        pl.BlockSpec((1, 3), lambda i: (0, 0)),
        pl.BlockSpec((3, 1), lambda i: (0, 0)),
        pl.BlockSpec((1, 1), lambda i: (0, 0)),
    ],
    out_specs=pl.BlockSpec((BR, 1), lambda i: (i, 0)),
    out_shape=jax.ShapeDtypeStruct((H, 1), jnp.float32),
)


def kernel(x, edge_index, W1, b1, W2, b2, W3, b3, W4, b4, W5, b5):
    x = x.astype(jnp.float32)
    padn = R_PAD * LN - E
    pad_rows = N + (jnp.arange(padn, dtype=jnp.int32) % 2048)
    ei = edge_index.astype(jnp.int32)
    srcp = jnp.concatenate([ei[0], pad_rows]).reshape(R_PAD, LN)
    dstp = jnp.concatenate([ei[1], pad_rows]).reshape(R_PAD, LN)
    srcp, dstp = lax.optimization_barrier((srcp, dstp))
    ones128 = jnp.ones((LN, 1), jnp.float32)
    z1 = jnp.zeros((H, 1), jnp.float32)
    z2 = jnp.zeros((H, 2), jnp.float32)
    z4 = jnp.zeros((H, 4), jnp.float32)

    xp = jnp.concatenate([x, jnp.zeros((H - N, 2), jnp.float32)], axis=0)
    degp = _deg_kernel(dstp, ones128, z1)
    dinv, y1 = _tc1(degp, xp)
    agg1p = _agg2(srcp, dstp, y1, z2)
    w2p = jnp.concatenate([W2, jnp.zeros((4, 1), jnp.float32)], axis=1)
    u = _tc2(agg1p, y1, dinv, W1, b1.reshape(1, 4), w2p)
    agg2p = _agg4(srcp, dstp, u, z4)
    o = _tc3(agg2p, u, dinv, b2.reshape(1, 3), W3, b3.reshape(1, 4),
             W4, b4.reshape(1, 3), W5, b5.reshape(1, 1))
    return o[:N, 0]


# speedup vs baseline: 1.1022x; 1.1022x over previous
import functools

import jax
import jax.numpy as jnp
from jax import lax
from jax.experimental import pallas as pl
from jax.experimental.pallas import tpu as pltpu
from jax.experimental.pallas import tpu_sc as plsc

N = 100000
E = 6400000
LN = 128
R = E // LN
NC, NS = 2, 16
NW = NC * NS
K = 16
C = 98
RPW = K * C
R_PAD = NW * RPW
H = 102400
STRIPE = H // NS
BR = 2048
GRID = H // BR

_mesh = plsc.VectorSubcoreMesh(
    core_axis_name="c", subcore_axis_name="s", num_cores=NC, num_subcores=NS
)
_sc_params = pltpu.CompilerParams(use_tc_tiling_on_sc=False)


@functools.partial(
    pl.kernel,
    out_type=jax.ShapeDtypeStruct((NC, H, 1), jnp.float32),
    mesh=_mesh,
    scratch_types=[
        pltpu.VMEM((K, LN), jnp.int32),
        pltpu.VMEM((K, LN), jnp.int32),
        pltpu.VMEM((LN, 1), jnp.float32),
        pltpu.VMEM_SHARED((H, 1), jnp.float32),
        pltpu.SemaphoreType.DMA,
        pltpu.SemaphoreType.DMA,
    ],
    compiler_params=_sc_params,
)
def _deg_kernel(dst_hbm, ones_hbm, zeros_hbm, out_hbm, didx_a, didx_b, ones_v,
                hist_sh, sem_a, sem_b):
    c = lax.axis_index("c")
    s = lax.axis_index("s")
    wid = c * NS + s
    sl = pl.ds(s * STRIPE, STRIPE)
    pltpu.sync_copy(ones_hbm, ones_v)
    pltpu.sync_copy(zeros_hbm.at[sl], hist_sh.at[sl])
    plsc.subcore_barrier()
    base = wid * RPW

    def _half(didx, sem, row0, drain):
        if drain:
            for j in range(K):
                pltpu.make_async_copy(ones_v, hist_sh.at[didx.at[j]], sem).wait()
        pltpu.sync_copy(dst_hbm.at[pl.ds(row0, K)], didx)
        for j in range(K):
            pltpu.async_copy(ones_v, hist_sh.at[didx.at[j]], sem, add=True)

    _half(didx_a, sem_a, base, False)
    _half(didx_b, sem_b, base + K, False)

    @pl.loop(1, C // 2)
    def _pair(pi):
        row0 = base + pi * (2 * K)
        _half(didx_a, sem_a, row0, True)
        _half(didx_b, sem_b, row0 + K, True)

    for j in range(K):
        pltpu.make_async_copy(ones_v, hist_sh.at[didx_a.at[j]], sem_a).wait()
    for j in range(K):
        pltpu.make_async_copy(ones_v, hist_sh.at[didx_b.at[j]], sem_b).wait()

    plsc.subcore_barrier()
    pltpu.sync_copy(hist_sh.at[sl], out_hbm.at[c, sl])


def _make_agg(F):
    @functools.partial(
        pl.kernel,
        out_type=jax.ShapeDtypeStruct((NC, H, F), jnp.float32),
        mesh=_mesh,
        scratch_types=[
            pltpu.VMEM((K, LN), jnp.int32),
            pltpu.VMEM((K, LN), jnp.int32),
            pltpu.VMEM((K, LN, F), jnp.float32),
            pltpu.VMEM((K, LN), jnp.int32),
            pltpu.VMEM((K, LN), jnp.int32),
            pltpu.VMEM((K, LN, F), jnp.float32),
            pltpu.VMEM_SHARED((H, F), jnp.float32),
            pltpu.VMEM_SHARED((H, F), jnp.float32),
            pltpu.SemaphoreType.DMA,
            pltpu.SemaphoreType.DMA,
            pltpu.SemaphoreType.DMA,
            pltpu.SemaphoreType.DMA,
        ],
        compiler_params=_sc_params,
    )
    def _agg(src_hbm, dst_hbm, y_hbm, zeros_hbm, out_hbm,
             sidx_a, didx_a, rows_a, sidx_b, didx_b, rows_b,
             y_sh, agg_sh, sem_ga, sem_sa, sem_gb, sem_sb):
        c = lax.axis_index("c")
        s = lax.axis_index("s")
        wid = c * NS + s
        sl = pl.ds(s * STRIPE, STRIPE)
        pltpu.sync_copy(y_hbm.at[sl], y_sh.at[sl])
        pltpu.sync_copy(zeros_hbm.at[sl], agg_sh.at[sl])
        plsc.subcore_barrier()
        base = wid * RPW

        def _half(sidx, didx, rows_v, sem_g, sem_s, row0, drain):
            if drain:
                for j in range(K):
                    pltpu.make_async_copy(rows_v.at[j], agg_sh.at[didx.at[j]], sem_s).wait()
            pltpu.sync_copy(src_hbm.at[pl.ds(row0, K)], sidx)
            pltpu.sync_copy(dst_hbm.at[pl.ds(row0, K)], didx)
            gd = [
                pltpu.async_copy(y_sh.at[sidx.at[j]], rows_v.at[j], sem_g)
                for j in range(K)
            ]
            for d in gd:
                d.wait()
            for j in range(K):
                pltpu.async_copy(rows_v.at[j], agg_sh.at[didx.at[j]], sem_s, add=True)

        _half(sidx_a, didx_a, rows_a, sem_ga, sem_sa, base, False)
        _half(sidx_b, didx_b, rows_b, sem_gb, sem_sb, base + K, False)

        @pl.loop(1, C // 2)
        def _pair(pi):
            row0 = base + pi * (2 * K)
            _half(sidx_a, didx_a, rows_a, sem_ga, sem_sa, row0, True)
            _half(sidx_b, didx_b, rows_b, sem_gb, sem_sb, row0 + K, True)

        for j in range(K):
            pltpu.make_async_copy(rows_a.at[j], agg_sh.at[didx_a.at[j]], sem_sa).wait()
        for j in range(K):
            pltpu.make_async_copy(rows_b.at[j], agg_sh.at[didx_b.at[j]], sem_sb).wait()

        plsc.subcore_barrier()
        pltpu.sync_copy(agg_sh.at[sl], out_hbm.at[c, sl])

    return _agg


_agg2 = _make_agg(2)
_agg4 = _make_agg(4)


def _tc1_body(degp_ref, x_ref, dinv_ref, y1_ref):
    d = degp_ref[...]
    deg = d[0] + d[1] + 1.0
    dinv = lax.rsqrt(deg)
    dinv_ref[...] = dinv
    y1_ref[...] = x_ref[...] * dinv


_tc1 = pl.pallas_call(
    _tc1_body,
    grid=(GRID,),
    in_specs=[
        pl.BlockSpec((NC, BR, 1), lambda i: (0, i, 0)),
        pl.BlockSpec((BR, 2), lambda i: (i, 0)),
    ],
    out_specs=[
        pl.BlockSpec((BR, 1), lambda i: (i, 0)),
        pl.BlockSpec((BR, 2), lambda i: (i, 0)),
    ],
    out_shape=[
        jax.ShapeDtypeStruct((H, 1), jnp.float32),
        jax.ShapeDtypeStruct((H, 2), jnp.float32),
    ],
)


def _tc2_body(aggp_ref, y1_ref, dinv_ref, w1_ref, b1_ref, w2_ref, u_ref):
    a = aggp_ref[...]
    a1 = a[0] + a[1] + y1_ref[...]
    dinv = dinv_ref[...]
    t1 = a1 * dinv
    w1 = w1_ref[...]
    h = t1[:, 0:1] * w1[0:1, :] + t1[:, 1:2] * w1[1:2, :] + b1_ref[...]
    h = jnp.maximum(h, 0.0)
    w2 = w2_ref[...]
    z = (h[:, 0:1] * w2[0:1, :] + h[:, 1:2] * w2[1:2, :]
         + h[:, 2:3] * w2[2:3, :] + h[:, 3:4] * w2[3:4, :])
    u = z * dinv
    i = pl.program_id(0)
    rows = i * BR + lax.broadcasted_iota(jnp.int32, (BR, 1), 0)
    u_ref[...] = jnp.where(rows < N, u, 0.0)


_tc2 = pl.pallas_call(
    _tc2_body,
    grid=(GRID,),
    in_specs=[
        pl.BlockSpec((NC, BR, 2), lambda i: (0, i, 0)),
        pl.BlockSpec((BR, 2), lambda i: (i, 0)),
        pl.BlockSpec((BR, 1), lambda i: (i, 0)),
        pl.BlockSpec((2, 4), lambda i: (0, 0)),
        pl.BlockSpec((1, 4), lambda i: (0, 0)),
        pl.BlockSpec((4, 4), lambda i: (0, 0)),
    ],
    out_specs=pl.BlockSpec((BR, 4), lambda i: (i, 0)),
    out_shape=jax.ShapeDtypeStruct((H, 4), jnp.float32),
)


def _tc3_body(aggp_ref, u_ref, dinv_ref, b2_ref, w3_ref, b3_ref, w4_ref,
              b4_ref, w5_ref, b5_ref, o_ref):
    a = aggp_ref[...]
    a2 = a[0] + a[1] + u_ref[...]
    t2 = a2[:, 0:3] * dinv_ref[...] + b2_ref[...]
    g = 1.0 / (1.0 + jnp.exp(-t2))
    w3 = w3_ref[...]
    h3 = (g[:, 0:1] * w3[0:1, :] + g[:, 1:2] * w3[1:2, :]
          + g[:, 2:3] * w3[2:3, :] + b3_ref[...])
    h3 = jnp.maximum(h3, 0.0)
    w4 = w4_ref[...]
    h4 = (h3[:, 0:1] * w4[0:1, :] + h3[:, 1:2] * w4[1:2, :]
          + h3[:, 2:3] * w4[2:3, :] + h3[:, 3:4] * w4[3:4, :] + b4_ref[...])
    h4 = jnp.maximum(h4, 0.0)
    w5 = w5_ref[...]
    o_ref[...] = (h4[:, 0:1] * w5[0:1, :] + h4[:, 1:2] * w5[1:2, :]
                  + h4[:, 2:3] * w5[2:3, :] + b5_ref[...])


_tc3 = pl.pallas_call(
    _tc3_body,
    grid=(GRID,),
    in_specs=[
        pl.BlockSpec((NC, BR, 4), lambda i: (0, i, 0)),
        pl.BlockSpec((BR, 4), lambda i: (i, 0)),
        pl.BlockSpec((BR, 1), lambda i: (i, 0)),
        pl.BlockSpec((1, 3), lambda i: (0, 0)),
        pl.BlockSpec((3, 4), lambda i: (0, 0)),
        pl.BlockSpec((1, 4), lambda i: (0, 0)),
        pl.BlockSpec((4, 3), lambda i: (0, 0)),
        pl.BlockSpec((1, 3), lambda i: (0, 0)),
        pl.BlockSpec((3, 1), lambda i: (0, 0)),
        pl.BlockSpec((1, 1), lambda i: (0, 0)),
    ],
    out_specs=pl.BlockSpec((BR, 1), lambda i: (i, 0)),
    out_shape=jax.ShapeDtypeStruct((H, 1), jnp.float32),
)


def kernel(x, edge_index, W1, b1, W2, b2, W3, b3, W4, b4, W5, b5):
    x = x.astype(jnp.float32)
    padn = R_PAD * LN - E
    pad_rows = N + (jnp.arange(padn, dtype=jnp.int32) % 2048)
    ei = edge_index.astype(jnp.int32)
    srcp = jnp.concatenate([ei[0], pad_rows]).reshape(R_PAD, LN)
    dstp = jnp.concatenate([ei[1], pad_rows]).reshape(R_PAD, LN)
    srcp, dstp = lax.optimization_barrier((srcp, dstp))
    ones128 = jnp.ones((LN, 1), jnp.float32)
    z1 = jnp.zeros((H, 1), jnp.float32)
    z2 = jnp.zeros((H, 2), jnp.float32)
    z4 = jnp.zeros((H, 4), jnp.float32)

    xp = jnp.concatenate([x, jnp.zeros((H - N, 2), jnp.float32)], axis=0)
    degp = _deg_kernel(dstp, ones128, z1)
    dinv, y1 = _tc1(degp, xp)
    agg1p = _agg2(srcp, dstp, y1, z2)
    w2p = jnp.concatenate([W2, jnp.zeros((4, 1), jnp.float32)], axis=1)
    u = _tc2(agg1p, y1, dinv, W1, b1.reshape(1, 4), w2p)
    agg2p = _agg4(srcp, dstp, u, z4)
    o = _tc3(agg2p, u, dinv, b2.reshape(1, 3), W3, b3.reshape(1, 4),
             W4, b4.reshape(1, 3), W5, b5.reshape(1, 1))
    return o[:N, 0]


# speedup vs baseline: 1.2051x; 1.0934x over previous
import functools

import jax
import jax.numpy as jnp
from jax import lax
from jax.experimental import pallas as pl
from jax.experimental.pallas import tpu as pltpu
from jax.experimental.pallas import tpu_sc as plsc

N = 100000
E = 6400000
LN = 128
R = E // LN
NC, NS = 2, 16
NW = NC * NS
K = 16
C = 98
RPW = K * C
R_PAD = NW * RPW
H = 102400
STRIPE = H // NS
BR = 2048
GRID = H // BR

_mesh = plsc.VectorSubcoreMesh(
    core_axis_name="c", subcore_axis_name="s", num_cores=NC, num_subcores=NS
)
_sc_params = pltpu.CompilerParams(use_tc_tiling_on_sc=False)


@functools.partial(
    pl.kernel,
    out_type=jax.ShapeDtypeStruct((NC, H, 1), jnp.float32),
    mesh=_mesh,
    scratch_types=[
        pltpu.VMEM((K, LN), jnp.int32),
        pltpu.VMEM((K, LN), jnp.int32),
        pltpu.VMEM((LN, 1), jnp.float32),
        pltpu.VMEM_SHARED((H, 1), jnp.float32),
        pltpu.SemaphoreType.DMA,
        pltpu.SemaphoreType.DMA,
    ],
    compiler_params=_sc_params,
)
def _deg_kernel(dst_hbm, ones_hbm, zeros_hbm, out_hbm, didx_a, didx_b, ones_v,
                hist_sh, sem_a, sem_b):
    c = lax.axis_index("c")
    s = lax.axis_index("s")
    wid = c * NS + s
    sl = pl.ds(s * STRIPE, STRIPE)
    pltpu.sync_copy(ones_hbm, ones_v)
    pltpu.sync_copy(zeros_hbm.at[sl], hist_sh.at[sl])
    plsc.subcore_barrier()
    base = wid * RPW

    def _half(didx, sem, row0, drain):
        if drain:
            for j in range(K):
                pltpu.make_async_copy(ones_v, hist_sh.at[didx.at[j]], sem).wait()
        pltpu.sync_copy(dst_hbm.at[pl.ds(row0, K)], didx)
        for j in range(K):
            pltpu.async_copy(ones_v, hist_sh.at[didx.at[j]], sem, add=True)

    _half(didx_a, sem_a, base, False)
    _half(didx_b, sem_b, base + K, False)

    @pl.loop(1, C // 2)
    def _pair(pi):
        row0 = base + pi * (2 * K)
        _half(didx_a, sem_a, row0, True)
        _half(didx_b, sem_b, row0 + K, True)

    for j in range(K):
        pltpu.make_async_copy(ones_v, hist_sh.at[didx_a.at[j]], sem_a).wait()
    for j in range(K):
        pltpu.make_async_copy(ones_v, hist_sh.at[didx_b.at[j]], sem_b).wait()

    plsc.subcore_barrier()
    pltpu.sync_copy(hist_sh.at[sl], out_hbm.at[c, sl])


def _make_agg(F):
    @functools.partial(
        pl.kernel,
        out_type=jax.ShapeDtypeStruct((NC, H, F), jnp.float32),
        mesh=_mesh,
        scratch_types=[
            pltpu.VMEM((K, LN), jnp.int32),
            pltpu.VMEM((K, LN), jnp.int32),
            pltpu.VMEM((K, LN, F), jnp.float32),
            pltpu.VMEM((K, LN), jnp.int32),
            pltpu.VMEM((K, LN), jnp.int32),
            pltpu.VMEM((K, LN, F), jnp.float32),
            pltpu.VMEM_SHARED((H, F), jnp.float32),
            pltpu.VMEM_SHARED((H, F), jnp.float32),
            pltpu.SemaphoreType.DMA,
            pltpu.SemaphoreType.DMA,
            pltpu.SemaphoreType.DMA,
            pltpu.SemaphoreType.DMA,
        ],
        compiler_params=_sc_params,
    )
    def _agg(src_hbm, dst_hbm, y_hbm, zeros_hbm, out_hbm,
             sidx_a, didx_a, rows_a, sidx_b, didx_b, rows_b,
             y_sh, agg_sh, sem_ga, sem_sa, sem_gb, sem_sb):
        c = lax.axis_index("c")
        s = lax.axis_index("s")
        wid = c * NS + s
        sl = pl.ds(s * STRIPE, STRIPE)
        pltpu.sync_copy(y_hbm.at[sl], y_sh.at[sl])
        pltpu.sync_copy(zeros_hbm.at[sl], agg_sh.at[sl])
        plsc.subcore_barrier()
        base = wid * RPW

        def _half(sidx, didx, rows_v, sem_g, sem_s, row0, drain):
            if drain:
                for j in range(K):
                    pltpu.make_async_copy(rows_v.at[j], agg_sh.at[didx.at[j]], sem_s).wait()
            pltpu.sync_copy(src_hbm.at[pl.ds(row0, K)], sidx)
            pltpu.sync_copy(dst_hbm.at[pl.ds(row0, K)], didx)
            gd = [
                pltpu.async_copy(y_sh.at[sidx.at[j]], rows_v.at[j], sem_g)
                for j in range(K)
            ]
            for d in gd:
                d.wait()
            for j in range(K):
                pltpu.async_copy(rows_v.at[j], agg_sh.at[didx.at[j]], sem_s, add=True)

        _half(sidx_a, didx_a, rows_a, sem_ga, sem_sa, base, False)
        _half(sidx_b, didx_b, rows_b, sem_gb, sem_sb, base + K, False)

        @pl.loop(1, C // 2)
        def _pair(pi):
            row0 = base + pi * (2 * K)
            _half(sidx_a, didx_a, rows_a, sem_ga, sem_sa, row0, True)
            _half(sidx_b, didx_b, rows_b, sem_gb, sem_sb, row0 + K, True)

        for j in range(K):
            pltpu.make_async_copy(rows_a.at[j], agg_sh.at[didx_a.at[j]], sem_sa).wait()
        for j in range(K):
            pltpu.make_async_copy(rows_b.at[j], agg_sh.at[didx_b.at[j]], sem_sb).wait()

        plsc.subcore_barrier()
        pltpu.sync_copy(agg_sh.at[sl], out_hbm.at[c, sl])

    return _agg


_agg2 = _make_agg(2)
_agg4 = _make_agg(4)


HP = H // 128
PBR = BR // 128

def _tc1_body(degp_ref, x_ref, dinv_ref, y1_ref):
    d = degp_ref[...]
    deg = d[0] + d[1] + 1.0
    dinv = lax.rsqrt(deg)
    dinv_ref[...] = dinv
    y1_ref[...] = x_ref[...] * dinv


_tc1 = pl.pallas_call(
    _tc1_body,
    grid=(GRID,),
    in_specs=[
        pl.BlockSpec((NC, PBR, 128), lambda i: (0, i, 0)),
        pl.BlockSpec((2, PBR, 128), lambda i: (0, i, 0)),
    ],
    out_specs=[
        pl.BlockSpec((PBR, 128), lambda i: (i, 0)),
        pl.BlockSpec((2, PBR, 128), lambda i: (0, i, 0)),
    ],
    out_shape=[
        jax.ShapeDtypeStruct((HP, 128), jnp.float32),
        jax.ShapeDtypeStruct((2, HP, 128), jnp.float32),
    ],
)


def _tc2_body(aggp_ref, y1_ref, dinv_ref, w1_ref, b1_ref, w2_ref, mask_ref, u_ref):
    a = aggp_ref[...]
    y1 = y1_ref[...]
    dinv = dinv_ref[...]
    t = [(a[0, k] + a[1, k] + y1[k]) * dinv for k in range(2)]
    h = [jnp.maximum(t[0] * w1_ref[0, j] + t[1] * w1_ref[1, j]
                     + b1_ref[0, j], 0.0)
         for j in range(4)]
    mask = mask_ref[...]
    md = mask * dinv
    for j in range(4):
        z = (h[0] * w2_ref[0, j] + h[1] * w2_ref[1, j]
             + h[2] * w2_ref[2, j] + h[3] * w2_ref[3, j])
        u_ref[j, :, :] = z * md


_tc2 = pl.pallas_call(
    _tc2_body,
    grid=(GRID,),
    in_specs=[
        pl.BlockSpec((NC, 2, PBR, 128), lambda i: (0, 0, i, 0)),
        pl.BlockSpec((2, PBR, 128), lambda i: (0, i, 0)),
        pl.BlockSpec((PBR, 128), lambda i: (i, 0)),
        pl.BlockSpec((2, 4), lambda i: (0, 0), memory_space=pltpu.SMEM),
        pl.BlockSpec((1, 4), lambda i: (0, 0), memory_space=pltpu.SMEM),
        pl.BlockSpec((4, 4), lambda i: (0, 0), memory_space=pltpu.SMEM),
        pl.BlockSpec((PBR, 128), lambda i: (i, 0)),
    ],
    out_specs=pl.BlockSpec((4, PBR, 128), lambda i: (0, i, 0)),
    out_shape=jax.ShapeDtypeStruct((4, HP, 128), jnp.float32),
)


def _tc3_body(aggp_ref, u_ref, dinv_ref, b2_ref, w3_ref, b3_ref, w4_ref,
              b4_ref, w5_ref, b5_ref, o_ref):
    a = aggp_ref[...]
    u = u_ref[...]
    dinv = dinv_ref[...]
    g = [1.0 / (1.0 + jnp.exp(-((a[0, k] + a[1, k] + u[k]) * dinv
                                + b2_ref[0, k])))
         for k in range(3)]
    h3 = [jnp.maximum(g[0] * w3_ref[0, j] + g[1] * w3_ref[1, j]
                      + g[2] * w3_ref[2, j] + b3_ref[0, j], 0.0)
          for j in range(4)]
    h4 = [jnp.maximum(h3[0] * w4_ref[0, j] + h3[1] * w4_ref[1, j]
                      + h3[2] * w4_ref[2, j] + h3[3] * w4_ref[3, j]
                      + b4_ref[0, j], 0.0) for j in range(3)]
    o_ref[...] = (h4[0] * w5_ref[0, 0] + h4[1] * w5_ref[1, 0]
                  + h4[2] * w5_ref[2, 0] + b5_ref[0, 0])


_tc3 = pl.pallas_call(
    _tc3_body,
    grid=(GRID,),
    in_specs=[
        pl.BlockSpec((NC, 4, PBR, 128), lambda i: (0, 0, i, 0)),
        pl.BlockSpec((4, PBR, 128), lambda i: (0, i, 0)),
        pl.BlockSpec((PBR, 128), lambda i: (i, 0)),
        pl.BlockSpec((1, 3), lambda i: (0, 0), memory_space=pltpu.SMEM),
        pl.BlockSpec((3, 4), lambda i: (0, 0), memory_space=pltpu.SMEM),
        pl.BlockSpec((1, 4), lambda i: (0, 0), memory_space=pltpu.SMEM),
        pl.BlockSpec((4, 3), lambda i: (0, 0), memory_space=pltpu.SMEM),
        pl.BlockSpec((1, 3), lambda i: (0, 0), memory_space=pltpu.SMEM),
        pl.BlockSpec((3, 1), lambda i: (0, 0), memory_space=pltpu.SMEM),
        pl.BlockSpec((1, 1), lambda i: (0, 0), memory_space=pltpu.SMEM),
    ],
    out_specs=pl.BlockSpec((PBR, 128), lambda i: (i, 0)),
    out_shape=jax.ShapeDtypeStruct((HP, 128), jnp.float32),
)


def kernel(x, edge_index, W1, b1, W2, b2, W3, b3, W4, b4, W5, b5):
    x = x.astype(jnp.float32)
    padn = R_PAD * LN - E
    pad_rows = N + (jnp.arange(padn, dtype=jnp.int32) % 2048)
    ei = edge_index.astype(jnp.int32)
    srcp = jnp.concatenate([ei[0], pad_rows]).reshape(R_PAD, LN)
    dstp = jnp.concatenate([ei[1], pad_rows]).reshape(R_PAD, LN)
    srcp, dstp = lax.optimization_barrier((srcp, dstp))
    ones128 = jnp.ones((LN, 1), jnp.float32)
    z1 = jnp.zeros((H, 1), jnp.float32)
    z2 = jnp.zeros((H, 2), jnp.float32)
    z4 = jnp.zeros((H, 4), jnp.float32)

    xpl = jnp.pad(x.T, ((0, 0), (0, H - N))).reshape(2, H // 128, 128)
    mask = jnp.pad(jnp.ones((N,), jnp.float32), (0, H - N)).reshape(H // 128, 128)

    degp = _deg_kernel(dstp, ones128, z1)
    degp_pl = degp.reshape(NC, H // 128, 128)
    dinv_pl, y1_pl = _tc1(degp_pl, xpl)
    y1 = y1_pl.reshape(2, H).T
    agg1p = _agg2(srcp, dstp, y1, z2)
    agg1p_pl = jnp.transpose(agg1p, (0, 2, 1)).reshape(NC, 2, H // 128, 128)
    w2p = jnp.concatenate([W2, jnp.zeros((4, 1), jnp.float32)], axis=1)
    u_pl = _tc2(agg1p_pl, y1_pl, dinv_pl, W1, b1.reshape(1, 4), w2p, mask)
    u = u_pl.reshape(4, H).T
    agg2p = _agg4(srcp, dstp, u, z4)
    agg2p_pl = jnp.transpose(agg2p, (0, 2, 1)).reshape(NC, 4, H // 128, 128)
    o = _tc3(agg2p_pl, u_pl, dinv_pl, b2.reshape(1, 3), W3, b3.reshape(1, 4),
             W4, b4.reshape(1, 3), W5, b5.reshape(1, 1))
    return o.reshape(H)[:N]
